# Initial kernel scaffold; baseline (speedup 1.0000x reference)
import jax
import jax.numpy as jnp
from jax.experimental import pallas as pl


def kernel(init_embed, init_rel, edge_norm, params, edge_index, edge_type, subj, rel):
    raise NotImplementedError("write your pallas kernel here")


# speedup vs baseline: 2.8620x; 2.8620x over previous
import functools

import jax
import jax.numpy as jnp
from jax import lax
from jax.experimental import pallas as pl
from jax.experimental.pallas import tpu as pltpu
from jax.experimental.pallas import tpu_sc as plsc

NUM_ENT = 10000
NUM_REL2 = 400
DIM = 128
N_EDGES = 320000
HALF = N_EDGES // 2
BATCH = 4096

NC, NS, L = 2, 16, 16
CHUNK = 128
EDGES_PER_TILE = 10240
CHUNKS_PER_TILE = EDGES_PER_TILE // CHUNK
HALF_PAD = EDGES_PER_TILE * NS
NUM_ENT_PAD = 10240
ROWS_PER_TILE = NUM_ENT_PAD // NS
GROUPS = CHUNK // L


def _edge_body(x_hbm, r_hbm, src_hbm, dst_hbm, et_hbm, nrm_hbm, zeros_hbm,
               out_hbm, acc, r_sp, src_v, dst_v, et_v, nrm_v, rows_v,
               r_rows_v, sem, sem2):
    c = lax.axis_index("c")
    s = lax.axis_index("s")
    pltpu.sync_copy(zeros_hbm, acc.at[pl.ds(s * ROWS_PER_TILE, ROWS_PER_TILE)])
    @pl.when(s == 0)
    def _():
        pltpu.sync_copy(r_hbm, r_sp)
    plsc.subcore_barrier()

    base_tile = c * HALF_PAD + s * EDGES_PER_TILE

    def chunk_body(j, carry):
        base = base_tile + j * CHUNK
        pltpu.sync_copy(src_hbm.at[pl.ds(base, CHUNK)], src_v)
        pltpu.sync_copy(dst_hbm.at[pl.ds(base, CHUNK)], dst_v)
        pltpu.sync_copy(et_hbm.at[pl.ds(base, CHUNK)], et_v)
        pltpu.sync_copy(nrm_hbm.at[pl.ds(base, CHUNK)], nrm_v)
        cx = pltpu.async_copy(x_hbm.at[src_v], rows_v, sem)
        cr = pltpu.async_copy(r_sp.at[et_v], r_rows_v, sem2)
        cx.wait()
        cr.wait()

        def group_body(g, carry2):
            nrm16 = nrm_v[pl.ds(g * L, L)]
            for l in range(L):
                nm = nrm16[l]
                e = g * L + l
                for cc in range(DIM // L):
                    xv = rows_v[e, pl.ds(cc * L, L)]
                    rv = r_rows_v[e, pl.ds(cc * L, L)]
                    rows_v[e, pl.ds(cc * L, L)] = xv * rv * nm
            return carry2

        lax.fori_loop(0, GROUPS, group_body, 0)
        pltpu.sync_copy(rows_v, acc.at[dst_v], add=True)
        return carry

    lax.fori_loop(0, CHUNKS_PER_TILE, chunk_body, 0)
    plsc.subcore_barrier()
    pltpu.sync_copy(acc.at[pl.ds(s * ROWS_PER_TILE, ROWS_PER_TILE)],
                    out_hbm.at[c, pl.ds(s * ROWS_PER_TILE, ROWS_PER_TILE)])


_edge_kernel = functools.partial(
    pl.kernel,
    _edge_body,
    out_type=jax.ShapeDtypeStruct((NC, NUM_ENT_PAD, DIM), jnp.float32),
    mesh=plsc.VectorSubcoreMesh(core_axis_name="c", subcore_axis_name="s",
                                num_cores=NC, num_subcores=NS),
    scratch_types=[
        pltpu.VMEM_SHARED((NUM_ENT_PAD, DIM), jnp.float32),
        pltpu.VMEM_SHARED((NUM_REL2, DIM), jnp.float32),
        pltpu.VMEM((CHUNK,), jnp.int32),
        pltpu.VMEM((CHUNK,), jnp.int32),
        pltpu.VMEM((CHUNK,), jnp.int32),
        pltpu.VMEM((CHUNK,), jnp.float32),
        pltpu.VMEM((CHUNK, DIM), jnp.float32),
        pltpu.VMEM((CHUNK, DIM), jnp.float32),
        pltpu.SemaphoreType.DMA,
        pltpu.SemaphoreType.DMA,
    ],
)()


def _dense_body(aggin_ref, aggout_ref, x_ref, r_ref, inw_ref, outw_ref,
                loopw_ref, looprel_ref, bias_ref, gamma_ref, beta_ref,
                wrel_ref, h_out, r_out):
    t = jnp.dot(aggin_ref[...], inw_ref[...], preferred_element_type=jnp.float32)
    t = t + jnp.dot(aggout_ref[...], outw_ref[...], preferred_element_type=jnp.float32)
    t = t + jnp.dot(x_ref[...] * looprel_ref[...], loopw_ref[...],
                    preferred_element_type=jnp.float32)
    h = t * (1.0 / 3.0) + bias_ref[...]
    mean = jnp.mean(h, axis=0, keepdims=True)
    var = jnp.mean(h * h, axis=0, keepdims=True) - mean * mean
    h = (h - mean) * lax.rsqrt(var + 1e-5) * gamma_ref[...] + beta_ref[...]
    h_out[...] = jnp.tanh(h)
    r_out[...] = jnp.dot(r_ref[...], wrel_ref[...],
                         preferred_element_type=jnp.float32)


_dense_kernel = pl.pallas_call(
    _dense_body,
    out_shape=[
        jax.ShapeDtypeStruct((NUM_ENT, DIM), jnp.float32),
        jax.ShapeDtypeStruct((NUM_REL2, DIM), jnp.float32),
    ],
)

BPW = BATCH // (NC * NS)


def _gather_body(x_hbm, r_hbm, subj_hbm, rel_hbm, sub_out, rel_out,
                 idx_v, rows_v, sem):
    c = lax.axis_index("c")
    s = lax.axis_index("s")
    base = (s * NC + c) * BPW
    pltpu.sync_copy(subj_hbm.at[pl.ds(base, BPW)], idx_v)
    pltpu.async_copy(x_hbm.at[idx_v], rows_v, sem).wait()
    pltpu.sync_copy(rows_v, sub_out.at[pl.ds(base, BPW)])
    pltpu.sync_copy(rel_hbm.at[pl.ds(base, BPW)], idx_v)
    pltpu.async_copy(r_hbm.at[idx_v], rows_v, sem).wait()
    pltpu.sync_copy(rows_v, rel_out.at[pl.ds(base, BPW)])


_gather_kernel = functools.partial(
    pl.kernel,
    _gather_body,
    out_type=[
        jax.ShapeDtypeStruct((BATCH, DIM), jnp.float32),
        jax.ShapeDtypeStruct((BATCH, DIM), jnp.float32),
    ],
    mesh=plsc.VectorSubcoreMesh(core_axis_name="c", subcore_axis_name="s",
                                num_cores=NC, num_subcores=NS),
    scratch_types=[
        pltpu.VMEM((BPW,), jnp.int32),
        pltpu.VMEM((BPW, DIM), jnp.float32),
        pltpu.SemaphoreType.DMA,
    ],
)()


def _pad_halves(a, fill):
    pad = jnp.full((HALF_PAD - HALF,), fill, a.dtype)
    return jnp.concatenate([a[:HALF], pad, a[HALF:], pad])


def kernel(init_embed, init_rel, edge_norm, params, edge_index, edge_type,
           subj, rel):
    src_f = _pad_halves(edge_index[0], 0)
    dst_f = _pad_halves(edge_index[1], 0)
    et_f = _pad_halves(edge_type, 0)
    nrm_f = _pad_halves(edge_norm, 0.0)
    zeros = jnp.zeros((ROWS_PER_TILE, DIM), jnp.float32)

    def layer(x, r, p):
        agg = _edge_kernel(x, r, src_f, dst_f, et_f, nrm_f, zeros)
        return _dense_kernel(
            agg[0, :NUM_ENT], agg[1, :NUM_ENT], x, r, p["in_w"], p["out_w"], p["loop_w"],
            p["loop_rel"], p["bias"].reshape(1, DIM),
            p["bn_gamma"].reshape(1, DIM), p["bn_beta"].reshape(1, DIM),
            p["w_rel"])

    h1, r1 = layer(init_embed, init_rel, params["layer1"])
    h2, r2 = layer(h1, r1, params["layer2"])
    sub_emb, rel_emb = _gather_kernel(h2, r2, subj, rel)
    return sub_emb, rel_emb, h2


# speedup vs baseline: 3.2973x; 1.1521x over previous
import functools

import jax
import jax.numpy as jnp
from jax import lax
from jax.experimental import pallas as pl
from jax.experimental.pallas import tpu as pltpu
from jax.experimental.pallas import tpu_sc as plsc

NUM_ENT = 10000
NUM_REL2 = 400
DIM = 128
N_EDGES = 320000
HALF = N_EDGES // 2
BATCH = 4096

NC, NS, L = 2, 16, 16
CHUNK = 64
EDGES_PER_TILE = 10240
CHUNKS_PER_TILE = EDGES_PER_TILE // CHUNK
HALF_PAD = EDGES_PER_TILE * NS
NUM_ENT_PAD = 10240
ROWS_PER_TILE = NUM_ENT_PAD // NS
GROUPS = CHUNK // L
NBLK = NC * NS * CHUNKS_PER_TILE


def _edge_body(x_hbm, r_hbm, eidx_hbm, nrm_hbm, zeros_hbm, out_hbm,
               acc, r_sp,
               idx0, idx1, nrm0, nrm1, rows0, rows1, rr0, rr1,
               sx0, sx1, sr0, sr1, ss0, ss1):
    c = lax.axis_index("c")
    s = lax.axis_index("s")
    pltpu.sync_copy(zeros_hbm, acc.at[pl.ds(s * ROWS_PER_TILE, ROWS_PER_TILE)])
    @pl.when(s == 0)
    def _():
        pltpu.sync_copy(r_hbm, r_sp)
    plsc.subcore_barrier()

    idx = (idx0, idx1)
    nrm = (nrm0, nrm1)
    rows = (rows0, rows1)
    rr = (rr0, rr1)
    sx = (sx0, sx1)
    sr = (sr0, sr1)
    ss = (ss0, ss1)
    blk_base = (c * NS + s) * CHUNKS_PER_TILE

    def stage(j, b):
        pltpu.sync_copy(eidx_hbm.at[blk_base + j], idx[b])
        pltpu.sync_copy(nrm_hbm.at[blk_base + j], nrm[b])
        pltpu.async_copy(x_hbm.at[idx[b].at[0]], rows[b], sx[b])
        pltpu.async_copy(r_sp.at[idx[b].at[2]], rr[b], sr[b])

    def wait_gathers(b):
        pltpu.make_async_copy(x_hbm.at[idx[b].at[0]], rows[b], sx[b]).wait()
        pltpu.make_async_copy(r_sp.at[idx[b].at[2]], rr[b], sr[b]).wait()

    def wait_scatter(b):
        pltpu.make_async_copy(rows[b], acc.at[idx[b].at[1]], ss[b]).wait()

    stage(0, 0)

    def pair_body(jj, carry):
        for b in (0, 1):
            j = jj * 2 + b
            wait_gathers(b)

            @pl.when(j >= 1)
            def _():
                wait_scatter(1 - b)

            @pl.when(j + 1 < CHUNKS_PER_TILE)
            def _():
                stage(j + 1, 1 - b)

            def group_body(g, carry2):
                nrm16 = nrm[b][pl.ds(g * L, L)]
                for l in range(L):
                    nm = nrm16[l]
                    e = g * L + l
                    for cc in range(DIM // L):
                        xv = rows[b][e, pl.ds(cc * L, L)]
                        rv = rr[b][e, pl.ds(cc * L, L)]
                        rows[b][e, pl.ds(cc * L, L)] = xv * rv * nm
                return carry2

            lax.fori_loop(0, GROUPS, group_body, 0)
            pltpu.async_copy(rows[b], acc.at[idx[b].at[1]], ss[b], add=True)
        return carry

    lax.fori_loop(0, CHUNKS_PER_TILE // 2, pair_body, 0)
    wait_scatter(1)
    plsc.subcore_barrier()
    pltpu.sync_copy(acc.at[pl.ds(s * ROWS_PER_TILE, ROWS_PER_TILE)],
                    out_hbm.at[c, pl.ds(s * ROWS_PER_TILE, ROWS_PER_TILE)])


_edge_kernel = functools.partial(
    pl.kernel,
    _edge_body,
    out_type=jax.ShapeDtypeStruct((NC, NUM_ENT_PAD, DIM), jnp.float32),
    mesh=plsc.VectorSubcoreMesh(core_axis_name="c", subcore_axis_name="s",
                                num_cores=NC, num_subcores=NS),
    scratch_types=(
        [
            pltpu.VMEM_SHARED((NUM_ENT_PAD, DIM), jnp.float32),
            pltpu.VMEM_SHARED((NUM_REL2, DIM), jnp.float32),
        ]
        + [pltpu.VMEM((3, CHUNK), jnp.int32)] * 2
        + [pltpu.VMEM((CHUNK,), jnp.float32)] * 2
        + [pltpu.VMEM((CHUNK, DIM), jnp.float32)] * 4
        + [pltpu.SemaphoreType.DMA] * 6
    ),
)()


def _dense_body(aggin_ref, aggout_ref, x_ref, r_ref, inw_ref, outw_ref,
                loopw_ref, looprel_ref, bias_ref, gamma_ref, beta_ref,
                wrel_ref, h_out, r_out):
    t = jnp.dot(aggin_ref[...], inw_ref[...], preferred_element_type=jnp.float32)
    t = t + jnp.dot(aggout_ref[...], outw_ref[...], preferred_element_type=jnp.float32)
    t = t + jnp.dot(x_ref[...] * looprel_ref[...], loopw_ref[...],
                    preferred_element_type=jnp.float32)
    h = t * (1.0 / 3.0) + bias_ref[...]
    mean = jnp.mean(h, axis=0, keepdims=True)
    var = jnp.mean(h * h, axis=0, keepdims=True) - mean * mean
    h = (h - mean) * lax.rsqrt(var + 1e-5) * gamma_ref[...] + beta_ref[...]
    h_out[...] = jnp.tanh(h)
    r_out[...] = jnp.dot(r_ref[...], wrel_ref[...],
                         preferred_element_type=jnp.float32)


_dense_kernel = pl.pallas_call(
    _dense_body,
    out_shape=[
        jax.ShapeDtypeStruct((NUM_ENT, DIM), jnp.float32),
        jax.ShapeDtypeStruct((NUM_REL2, DIM), jnp.float32),
    ],
)

BPW = BATCH // (NC * NS)


def _gather_body(x_hbm, r_hbm, subj_hbm, rel_hbm, sub_out, rel_out,
                 idx_v, rows_v, sem):
    c = lax.axis_index("c")
    s = lax.axis_index("s")
    base = (s * NC + c) * BPW
    pltpu.sync_copy(subj_hbm.at[pl.ds(base, BPW)], idx_v)
    pltpu.async_copy(x_hbm.at[idx_v], rows_v, sem).wait()
    pltpu.sync_copy(rows_v, sub_out.at[pl.ds(base, BPW)])
    pltpu.sync_copy(rel_hbm.at[pl.ds(base, BPW)], idx_v)
    pltpu.async_copy(r_hbm.at[idx_v], rows_v, sem).wait()
    pltpu.sync_copy(rows_v, rel_out.at[pl.ds(base, BPW)])


_gather_kernel = functools.partial(
    pl.kernel,
    _gather_body,
    out_type=[
        jax.ShapeDtypeStruct((BATCH, DIM), jnp.float32),
        jax.ShapeDtypeStruct((BATCH, DIM), jnp.float32),
    ],
    mesh=plsc.VectorSubcoreMesh(core_axis_name="c", subcore_axis_name="s",
                                num_cores=NC, num_subcores=NS),
    scratch_types=[
        pltpu.VMEM((BPW,), jnp.int32),
        pltpu.VMEM((BPW, DIM), jnp.float32),
        pltpu.SemaphoreType.DMA,
    ],
)()


def _pad_halves(a, fill):
    pad = jnp.full((HALF_PAD - HALF,), fill, a.dtype)
    return jnp.concatenate([a[:HALF], pad, a[HALF:], pad])


def _pack_edges(edge_index, edge_type, edge_norm):
    src_f = _pad_halves(edge_index[0], 0)
    dst_f = _pad_halves(edge_index[1], 0)
    et_f = _pad_halves(edge_type, 0)
    nrm_f = _pad_halves(edge_norm, 0.0)
    eidx = (jnp.stack([src_f, dst_f, et_f])
            .reshape(3, NBLK, CHUNK).transpose(1, 0, 2))
    return eidx, nrm_f.reshape(NBLK, CHUNK)


def kernel(init_embed, init_rel, edge_norm, params, edge_index, edge_type,
           subj, rel):
    eidx, nrm_blk = _pack_edges(edge_index, edge_type, edge_norm)
    zeros = jnp.zeros((ROWS_PER_TILE, DIM), jnp.float32)

    def layer(x, r, p):
        agg = _edge_kernel(x, r, eidx, nrm_blk, zeros)
        return _dense_kernel(
            agg[0, :NUM_ENT], agg[1, :NUM_ENT], x, r, p["in_w"], p["out_w"], p["loop_w"],
            p["loop_rel"], p["bias"].reshape(1, DIM),
            p["bn_gamma"].reshape(1, DIM), p["bn_beta"].reshape(1, DIM),
            p["w_rel"])

    h1, r1 = layer(init_embed, init_rel, params["layer1"])
    h2, r2 = layer(h1, r1, params["layer2"])
    sub_emb, rel_emb = _gather_kernel(h2, r2, subj, rel)
    return sub_emb, rel_emb, h2
